# baseline (speedup 1.0000x reference)
import functools

import jax
import jax.numpy as jnp
from jax import lax
from jax.experimental import pallas as pl
from jax.experimental.pallas import tpu as pltpu
from jax.experimental.pallas import tpu_sc as plsc

VOCAB = 100000
EMBED = 64
SEQ = 200
BATCH = 4096
MAXLEN = 512

NC, NS = 2, 16
NW = NC * NS
BC = BATCH // NW
LANES = 16
NCH = EMBED // LANES
TPAD = BC + 1
SB = 2
NG = SEQ // SB
NBUF = 2
NTV = 4

_mesh = plsc.VectorSubcoreMesh(
    core_axis_name="c", subcore_axis_name="s", num_cores=NC, num_subcores=NS
)


@functools.partial(
    pl.kernel,
    out_type=jax.ShapeDtypeStruct((SEQ, 8, NW, 8, BC), jnp.float32),
    mesh=_mesh,
    scratch_types=[
        pltpu.VMEM((NG, SB * BC), jnp.int32),
        pltpu.VMEM((SB * BC, EMBED), jnp.float32),
        pltpu.VMEM((SB * BC, EMBED), jnp.float32),
        pltpu.VMEM((8, 8, TPAD), jnp.float32),
        pltpu.VMEM((8, 8, TPAD), jnp.float32),
        pltpu.VMEM((8, 8, TPAD), jnp.float32),
        pltpu.VMEM((8, 8, TPAD), jnp.float32),
        pltpu.VMEM((BC, LANES), jnp.int32),
        pltpu.VMEM((SEQ, EMBED), jnp.float32),
        pltpu.SemaphoreType.DMA,
        pltpu.SemaphoreType.DMA,
        pltpu.SemaphoreType.DMA,
        pltpu.SemaphoreType.DMA,
        pltpu.SemaphoreType.DMA,
        pltpu.SemaphoreType.DMA,
    ],
    compiler_params=pltpu.CompilerParams(
        use_tc_tiling_on_sc=False, needs_layout_passes=False
    ),
)
def _embed_kernel(
    xt_hbm, tab_hbm, pos_hbm, out_hbm,
    idx_all, rows0, rows1, tv0, tv1, tv2, tv3, bmat, pos_v,
    gsem0, gsem1, wsem0, wsem1, wsem2, wsem3,
):
    wid = lax.axis_index("s") * NC + lax.axis_index("c")
    pltpu.sync_copy(xt_hbm.at[wid], idx_all)
    pltpu.sync_copy(pos_hbm.at[pl.ds(0, SEQ)], pos_v)
    rows = (rows0, rows1)
    gsems = (gsem0, gsem1)
    tvs = (tv0, tv1, tv2, tv3)
    wsems = (wsem0, wsem1, wsem2, wsem3)
    dt_vecs = [(lax.iota(jnp.int32, LANES) + c * LANES) // 8 for c in range(NCH)]
    di_vecs = [(lax.iota(jnp.int32, LANES) + c * LANES) % 8 for c in range(NCH)]

    def binit(b, inner):
        bmat[b] = jnp.full((LANES,), b, dtype=jnp.int32)
        return inner

    lax.fori_loop(0, BC, binit, 0)

    def issue(g, p):
        pltpu.async_copy(tab_hbm.at[idx_all.at[g]], rows[p], gsems[p])

    def wait_gather(g, p):
        pltpu.make_async_copy(tab_hbm.at[idx_all.at[g]], rows[p], gsems[p]).wait()

    def wait_write(tp):
        pltpu.make_async_copy(
            tvs[tp].at[:, :, pl.ds(0, BC)], out_hbm.at[0, :, wid], wsems[tp]
        ).wait()

    def process(g, p, gpar):
        rows_v = rows[p]
        for j in range(SB):
            s = g * SB + j
            tp = 2 * gpar + j
            t_v = tvs[tp]
            pvecs = [pos_v[s, pl.ds(c * LANES, LANES)] for c in range(NCH)]

            def b_body(b, inner):
                bvec = bmat[b]
                for c in range(NCH):
                    val = rows_v[j * BC + b, pl.ds(c * LANES, LANES)] + pvecs[c]
                    plsc.store_scatter(t_v, [dt_vecs[c], di_vecs[c], bvec], val)
                return inner

            lax.fori_loop(0, BC, b_body, 0, unroll=8)
            pltpu.async_copy(
                t_v.at[:, :, pl.ds(0, BC)], out_hbm.at[s, :, wid], wsems[tp]
            )

    issue(0, 0)
    issue(1, 1)

    def loop_body(i, carry):
        for p in range(NBUF):
            g = i * NBUF + p
            wait_gather(g, p)

            @pl.when(g >= 2)
            def _():
                wait_write(2 * p)
                wait_write(2 * p + 1)

            process(g, p, p)

            @pl.when(g < NG - NBUF)
            def _():
                issue(g + NBUF, p)

        return carry

    lax.fori_loop(0, NG // NBUF, loop_body, 0)
    for tp in range(NTV):
        wait_write(tp)


def kernel(x, token_embed, pos_embed):
    xt = jnp.transpose(x.astype(jnp.int32))
    xprep = (
        xt.reshape(SEQ, NW, BC)
        .transpose(1, 0, 2)
        .reshape(NW, NG, SB * BC)
    )
    pos2d = pos_embed.reshape(MAXLEN, EMBED)
    out5 = _embed_kernel(xprep, token_embed, pos2d)
    return out5.transpose(2, 4, 0, 1, 3).reshape(BATCH, SEQ, EMBED)
